# baseline (speedup 1.0000x reference)
import functools
import math

import jax
import jax.numpy as jnp
from jax import lax
from jax.experimental import pallas as pl
from jax.experimental.pallas import tpu as pltpu
from jax.experimental.pallas import tpu_sc as plsc

D = 128
SCALE = math.sqrt(float(D))

NC = 2
NS = 16
NW = NC * NS
C = 64
NBUF = 2
LANES = 16


def _scale_rows(src, dst):
    def body(r, _):
        for l in range(D // LANES):
            off = l * LANES
            dst[r, pl.ds(off, LANES)] = src[r, pl.ds(off, LANES)] * SCALE
        return 0

    lax.fori_loop(0, C, body, 0)


def _make_emb(B, NCH):
    NG = NCH // NBUF
    mesh = plsc.VectorSubcoreMesh(core_axis_name="c", subcore_axis_name="s")

    @functools.partial(
        pl.kernel,
        mesh=mesh,
        out_type=jax.ShapeDtypeStruct((B, D), jnp.float32),
        scratch_types=[
            pltpu.VMEM((NCH * C,), jnp.int32),
            pltpu.VMEM((NBUF, C, D), jnp.float32),
            pltpu.VMEM((NBUF, C, D), jnp.float32),
            pltpu.SemaphoreType.DMA,
            pltpu.SemaphoreType.DMA,
            pltpu.SemaphoreType.DMA,
            pltpu.SemaphoreType.DMA,
        ],
    )
    def emb(table_hbm, idx_hbm, out_hbm, idx_v, g_ref, s_ref, gs0, gs1, ss0, ss1):
        cid = lax.axis_index("c")
        sid = lax.axis_index("s")
        wid = sid * NC + cid
        base_row = wid * (NCH * C)

        pltpu.sync_copy(idx_hbm.at[wid], idx_v)

        gsems = (gs0, gs1)
        ssems = (ss0, ss1)

        def gather_start(c, b):
            pltpu.make_async_copy(
                table_hbm.at[idx_v.at[pl.ds(c * C, C)]], g_ref.at[b], gsems[b]
            ).start()

        def gather_wait(c, b):
            pltpu.make_async_copy(
                table_hbm.at[idx_v.at[pl.ds(c * C, C)]], g_ref.at[b], gsems[b]
            ).wait()

        def scatter_start(c, b):
            pltpu.make_async_copy(
                s_ref.at[b], out_hbm.at[pl.ds(base_row + c * C, C)], ssems[b]
            ).start()

        def scatter_wait(c, b):
            pltpu.make_async_copy(
                s_ref.at[b], out_hbm.at[pl.ds(base_row + c * C, C)], ssems[b]
            ).wait()

        for b in range(NBUF):
            gather_start(b, b)

        def main(gi, _):
            for b in range(NBUF):
                c = gi * NBUF + b
                gather_wait(c, b)
                pl.when(gi >= 1)(lambda: scatter_wait(c - NBUF, b))
                _scale_rows(g_ref.at[b], s_ref.at[b])
                scatter_start(c, b)
                pl.when(gi <= NG - 2)(lambda: gather_start(c + NBUF, b))
            return 0

        lax.fori_loop(0, NG, main, 0)

        for b in range(NBUF):
            scatter_wait(NCH - NBUF + b, b)

    return emb


def kernel(tokens, table):
    n, t = tokens.shape
    B = n * t
    NCH = B // (NW * C)
    idx = tokens.reshape(-1).astype(jnp.int32).reshape(NW, NCH * C)
    out = _make_emb(B, NCH)(table, idx)
    return out.reshape(n, t, D)


# speedup vs baseline: 1.1336x; 1.1336x over previous
import functools
import math

import jax
import jax.numpy as jnp
from jax import lax
from jax.experimental import pallas as pl
from jax.experimental.pallas import tpu as pltpu
from jax.experimental.pallas import tpu_sc as plsc

D = 128
SCALE = math.sqrt(float(D))

NC = 2
NS = 16
NW = NC * NS
C = 128
LANES = 16


def _scale_rows(src, dst):
    def body(r, _):
        for l in range(D // LANES):
            off = l * LANES
            dst[r, pl.ds(off, LANES)] = src[r, pl.ds(off, LANES)] * SCALE
        return 0

    lax.fori_loop(0, C, body, 0)


def _make_emb(B, NCH):
    NP = NCH // 2
    mesh = plsc.VectorSubcoreMesh(core_axis_name="c", subcore_axis_name="s")

    @functools.partial(
        pl.kernel,
        mesh=mesh,
        out_type=jax.ShapeDtypeStruct((B, D), jnp.float32),
        scratch_types=[
            pltpu.VMEM((NCH * C,), jnp.int32),
            pltpu.VMEM((2, C, D), jnp.float32),
            pltpu.VMEM((2 * C, D), jnp.float32),
            pltpu.VMEM((2 * C, D), jnp.float32),
            pltpu.SemaphoreType.DMA,
            pltpu.SemaphoreType.DMA,
            pltpu.SemaphoreType.DMA,
            pltpu.SemaphoreType.DMA,
        ],
    )
    def emb(table_hbm, idx_hbm, out_hbm, idx_v, g_ref, sa_ref, sb_ref,
            gs0, gs1, ss0, ss1):
        cid = lax.axis_index("c")
        sid = lax.axis_index("s")
        wid = sid * NC + cid
        base_row = wid * (NCH * C)

        pltpu.sync_copy(idx_hbm.at[wid], idx_v)

        gsems = (gs0, gs1)
        srefs = (sa_ref, sb_ref)
        ssems = (ss0, ss1)

        def gather_start(c, b):
            pltpu.make_async_copy(
                table_hbm.at[idx_v.at[pl.ds(c * C, C)]], g_ref.at[b], gsems[b]
            ).start()

        def gather_wait(c, b):
            pltpu.make_async_copy(
                table_hbm.at[idx_v.at[pl.ds(c * C, C)]], g_ref.at[b], gsems[b]
            ).wait()

        def scatter_start(p, sp):
            pltpu.make_async_copy(
                srefs[sp],
                out_hbm.at[pl.ds(base_row + p * 2 * C, 2 * C)],
                ssems[sp],
            ).start()

        def scatter_wait(p, sp):
            pltpu.make_async_copy(
                srefs[sp],
                out_hbm.at[pl.ds(base_row + p * 2 * C, 2 * C)],
                ssems[sp],
            ).wait()

        def pair_body(p, sp, traced):
            gather_wait(2 * p, 0)
            if traced:
                pl.when(p >= 2)(lambda: scatter_wait(p - 2, sp))
            elif p >= 2:
                scatter_wait(p - 2, sp)
            _scale_rows(g_ref.at[0], srefs[sp].at[pl.ds(0, C)])
            if traced:
                pl.when(p <= NP - 2)(lambda: gather_start(2 * p + 2, 0))
            elif p <= NP - 2:
                gather_start(2 * p + 2, 0)
            gather_wait(2 * p + 1, 1)
            _scale_rows(g_ref.at[1], srefs[sp].at[pl.ds(C, C)])
            if traced:
                pl.when(p <= NP - 2)(lambda: gather_start(2 * p + 3, 1))
            elif p <= NP - 2:
                gather_start(2 * p + 3, 1)
            scatter_start(p, sp)

        gather_start(0, 0)
        gather_start(1, 1)

        def main(go, _):
            pair_body(go * 2, 0, True)
            pair_body(go * 2 + 1, 1, True)
            return 0

        lax.fori_loop(0, NP // 2, main, 0)
        if NP % 2:
            pair_body(NP - 1, (NP - 1) % 2, False)

        scatter_wait(NP - 2, (NP - 2) % 2)
        scatter_wait(NP - 1, (NP - 1) % 2)

    return emb


def kernel(tokens, table):
    n, t = tokens.shape
    B = n * t
    NCH = B // (NW * C)
    idx = tokens.reshape(-1).astype(jnp.int32).reshape(NW, NCH * C)
    out = _make_emb(B, NCH)(table, idx)
    return out.reshape(n, t, D)


# speedup vs baseline: 1.1363x; 1.0024x over previous
import functools
import math

import jax
import jax.numpy as jnp
from jax import lax
from jax.experimental import pallas as pl
from jax.experimental.pallas import tpu as pltpu
from jax.experimental.pallas import tpu_sc as plsc

D = 128
SCALE = math.sqrt(float(D))

NC = 2
NS = 16
NW = NC * NS
C = 128
NBUF = 2
LANES = 16


def _scale_rows(src, dst):
    def body(r, _):
        for l in range(D // LANES):
            off = l * LANES
            dst[r, pl.ds(off, LANES)] = src[r, pl.ds(off, LANES)] * SCALE
        return 0

    lax.fori_loop(0, C, body, 0)


def _make_emb(B, NCH):
    NG = NCH // NBUF
    mesh = plsc.VectorSubcoreMesh(core_axis_name="c", subcore_axis_name="s")

    @functools.partial(
        pl.kernel,
        mesh=mesh,
        out_type=jax.ShapeDtypeStruct((B, D), jnp.float32),
        scratch_types=[
            pltpu.VMEM((NCH * C,), jnp.int32),
            pltpu.VMEM((NBUF, C, D), jnp.float32),
            pltpu.VMEM((NBUF, C, D), jnp.float32),
            pltpu.SemaphoreType.DMA,
            pltpu.SemaphoreType.DMA,
            pltpu.SemaphoreType.DMA,
            pltpu.SemaphoreType.DMA,
        ],
    )
    def emb(table_hbm, idx_hbm, out_hbm, idx_v, g_ref, s_ref, gs0, gs1, ss0, ss1):
        cid = lax.axis_index("c")
        sid = lax.axis_index("s")
        wid = sid * NC + cid
        base_row = wid * (NCH * C)

        pltpu.sync_copy(idx_hbm.at[wid], idx_v)

        gsems = (gs0, gs1)
        ssems = (ss0, ss1)

        def gather_start(c, b):
            pltpu.make_async_copy(
                table_hbm.at[idx_v.at[pl.ds(c * C, C)]], g_ref.at[b], gsems[b]
            ).start()

        def gather_wait(c, b):
            pltpu.make_async_copy(
                table_hbm.at[idx_v.at[pl.ds(c * C, C)]], g_ref.at[b], gsems[b]
            ).wait()

        def scatter_start(c, b):
            pltpu.make_async_copy(
                s_ref.at[b], out_hbm.at[pl.ds(base_row + c * C, C)], ssems[b]
            ).start()

        def scatter_wait(c, b):
            pltpu.make_async_copy(
                s_ref.at[b], out_hbm.at[pl.ds(base_row + c * C, C)], ssems[b]
            ).wait()

        for b in range(NBUF):
            gather_start(b, b)

        def main(gi, _):
            for b in range(NBUF):
                c = gi * NBUF + b
                gather_wait(c, b)
                pl.when(gi >= 1)(lambda: scatter_wait(c - NBUF, b))
                _scale_rows(g_ref.at[b], s_ref.at[b])
                scatter_start(c, b)
                pl.when(gi <= NG - 2)(lambda: gather_start(c + NBUF, b))
            return 0

        lax.fori_loop(0, NG, main, 0)

        for b in range(NBUF):
            scatter_wait(NCH - NBUF + b, b)

    return emb


def kernel(tokens, table):
    n, t = tokens.shape
    B = n * t
    NCH = B // (NW * C)
    idx = tokens.reshape(-1).astype(jnp.int32).reshape(NW, NCH * C)
    out = _make_emb(B, NCH)(table, idx)
    return out.reshape(n, t, D)


# speedup vs baseline: 1.1460x; 1.0085x over previous
import functools
import math

import jax
import jax.numpy as jnp
from jax import lax
from jax.experimental import pallas as pl
from jax.experimental.pallas import tpu as pltpu
from jax.experimental.pallas import tpu_sc as plsc

D = 128
SCALE = math.sqrt(float(D))

NC = 2
NS = 16
NW = NC * NS
C = 128
CB = 256
LANES = 16


def _scale_rows(src, dst):
    def body(r, _):
        for l in range(D // LANES):
            off = l * LANES
            dst[r, pl.ds(off, LANES)] = src[r, pl.ds(off, LANES)] * SCALE
        return 0

    lax.fori_loop(0, C, body, 0)


def _make_emb(B, NI):
    NK = NI // CB
    mesh = plsc.VectorSubcoreMesh(core_axis_name="c", subcore_axis_name="s")

    @functools.partial(
        pl.kernel,
        mesh=mesh,
        out_type=jax.ShapeDtypeStruct((B, D), jnp.float32),
        scratch_types=[
            pltpu.VMEM((NI,), jnp.int32),
            pltpu.VMEM((2, CB, D), jnp.float32),
            pltpu.VMEM((2, C, D), jnp.float32),
            pltpu.SemaphoreType.DMA,
            pltpu.SemaphoreType.DMA,
            pltpu.SemaphoreType.DMA,
            pltpu.SemaphoreType.DMA,
        ],
    )
    def emb(table_hbm, idx_hbm, out_hbm, idx_v, g_ref, s_ref, gs0, gs1, ss0, ss1):
        cid = lax.axis_index("c")
        sid = lax.axis_index("s")
        wid = sid * NC + cid
        base_row = wid * NI

        pltpu.sync_copy(idx_hbm.at[wid], idx_v)

        gsems = (gs0, gs1)
        ssems = (ss0, ss1)

        def gather_start(k, kb):
            pltpu.make_async_copy(
                table_hbm.at[idx_v.at[pl.ds(k * CB, CB)]], g_ref.at[kb], gsems[kb]
            ).start()

        def gather_wait(k, kb):
            pltpu.make_async_copy(
                table_hbm.at[idx_v.at[pl.ds(k * CB, CB)]], g_ref.at[kb], gsems[kb]
            ).wait()

        def scatter_start(k, h):
            pltpu.make_async_copy(
                s_ref.at[h],
                out_hbm.at[pl.ds(base_row + k * CB + h * C, C)],
                ssems[h],
            ).start()

        def scatter_wait(k, h):
            pltpu.make_async_copy(
                s_ref.at[h],
                out_hbm.at[pl.ds(base_row + k * CB + h * C, C)],
                ssems[h],
            ).wait()

        def big_body(k, kb, wait_sc, issue_next):
            gather_wait(k, kb)
            for h in range(2):
                if wait_sc is None:
                    scatter_wait(k - 1, h)
                elif wait_sc is not False:
                    pl.when(wait_sc)(lambda: scatter_wait(k - 1, h))
                _scale_rows(g_ref.at[kb].at[pl.ds(h * C, C)], s_ref.at[h])
                scatter_start(k, h)
            if issue_next is None:
                gather_start(k + 2, kb)
            elif issue_next is not False:
                pl.when(issue_next)(lambda: gather_start(k + 2, kb))

        gather_start(0, 0)
        gather_start(1, 1)

        def main(gi, _):
            big_body(2 * gi, 0, gi >= 1, None)
            big_body(2 * gi + 1, 1, None, gi <= NK // 2 - 2)
            return 0

        lax.fori_loop(0, NK // 2, main, 0)
        big_body(NK - 1, (NK - 1) % 2, None, False)

        scatter_wait(NK - 1, 0)
        scatter_wait(NK - 1, 1)

    return emb


def kernel(tokens, table):
    n, t = tokens.shape
    B = n * t
    NI = B // NW
    idx = tokens.reshape(-1).astype(jnp.int32).reshape(NW, NI)
    out = _make_emb(B, NI)(table, idx)
    return out.reshape(n, t, D)
